# Initial kernel scaffold; baseline (speedup 1.0000x reference)
import jax
import jax.numpy as jnp
from jax.experimental import pallas as pl


def kernel(tokens, W_E):
    raise NotImplementedError("write your pallas kernel here")


# speedup vs baseline: 1.6843x; 1.6843x over previous
import functools

import jax
import jax.numpy as jnp
from jax import lax
from jax.experimental import pallas as pl
from jax.experimental.pallas import tpu as pltpu
from jax.experimental.pallas import tpu_sc as plsc

D_MODEL = 64
CHUNK = 128


@functools.partial(jax.jit, static_argnums=(2, 3))
def _embed_gather(idx2d, table, num_workers, chunks_per_w):
    n_rows = idx2d.shape[0] * idx2d.shape[1]
    mesh = plsc.VectorSubcoreMesh(core_axis_name="c", subcore_axis_name="s")
    num_cores = mesh.num_cores

    @functools.partial(
        pl.kernel,
        out_type=jax.ShapeDtypeStruct((n_rows, D_MODEL), jnp.float32),
        mesh=mesh,
        scratch_types=[
            pltpu.VMEM((chunks_per_w, CHUNK), jnp.int32),
            pltpu.VMEM((CHUNK, D_MODEL), jnp.float32),
            pltpu.SemaphoreType.DMA,
        ],
        compiler_params=pltpu.CompilerParams(use_tc_tiling_on_sc=False),
    )
    def k(idx_hbm, table_hbm, out_hbm, idx_v, rows_v, sem):
        wid = lax.axis_index("s") * num_cores + lax.axis_index("c")
        chunk0 = wid * chunks_per_w
        pltpu.sync_copy(idx_hbm.at[pl.ds(chunk0, chunks_per_w)], idx_v)

        @pl.loop(0, chunks_per_w)
        def _(j):
            pltpu.async_copy(table_hbm.at[idx_v.at[j]], rows_v, sem).wait()
            pltpu.sync_copy(rows_v, out_hbm.at[pl.ds((chunk0 + j) * CHUNK, CHUNK)])

    return k(idx2d, table)


def kernel(tokens, W_E):
    b, t = tokens.shape
    n_rows = b * t
    num_workers = 32
    assert n_rows % (num_workers * CHUNK) == 0
    chunks_per_w = n_rows // (num_workers * CHUNK)
    idx2d = tokens.reshape(n_rows // CHUNK, CHUNK).astype(jnp.int32)
    out = _embed_gather(idx2d, W_E, num_workers, chunks_per_w)
    return out.reshape(b, t, W_E.shape[1])


# speedup vs baseline: 1.8602x; 1.1044x over previous
import functools

import jax
import jax.numpy as jnp
from jax import lax
from jax.experimental import pallas as pl
from jax.experimental.pallas import tpu as pltpu
from jax.experimental.pallas import tpu_sc as plsc

D_MODEL = 64
CHUNK = 128
NBUF = 8


@functools.partial(jax.jit, static_argnums=(2, 3))
def _embed_gather(idx2d, table, num_workers, chunks_per_w):
    n_rows = idx2d.shape[0] * idx2d.shape[1]
    mesh = plsc.VectorSubcoreMesh(core_axis_name="c", subcore_axis_name="s")
    num_cores = mesh.num_cores
    n_groups = chunks_per_w // NBUF

    @functools.partial(
        pl.kernel,
        out_type=jax.ShapeDtypeStruct((n_rows, D_MODEL), jnp.float32),
        mesh=mesh,
        scratch_types=[
            pltpu.VMEM((chunks_per_w, CHUNK), jnp.int32),
            pltpu.VMEM((NBUF * CHUNK, D_MODEL), jnp.float32),
            pltpu.SemaphoreType.DMA,
        ],
        compiler_params=pltpu.CompilerParams(use_tc_tiling_on_sc=False),
    )
    def k(idx_hbm, table_hbm, out_hbm, idx_v, rows_v, sem):
        wid = lax.axis_index("s") * num_cores + lax.axis_index("c")
        chunk0 = wid * chunks_per_w
        pltpu.sync_copy(idx_hbm.at[pl.ds(chunk0, chunks_per_w)], idx_v)

        @pl.loop(0, n_groups)
        def _(g):
            descs = [
                pltpu.async_copy(
                    table_hbm.at[idx_v.at[g * NBUF + b]],
                    rows_v.at[pl.ds(b * CHUNK, CHUNK)],
                    sem,
                )
                for b in range(NBUF)
            ]
            for d in descs:
                d.wait()
            pltpu.sync_copy(
                rows_v, out_hbm.at[pl.ds((chunk0 + g * NBUF) * CHUNK, NBUF * CHUNK)]
            )

    return k(idx2d, table)


def kernel(tokens, W_E):
    b, t = tokens.shape
    n_rows = b * t
    num_workers = 32
    assert n_rows % (num_workers * CHUNK) == 0
    chunks_per_w = n_rows // (num_workers * CHUNK)
    idx2d = tokens.reshape(n_rows // CHUNK, CHUNK).astype(jnp.int32)
    out = _embed_gather(idx2d, W_E, num_workers, chunks_per_w)
    return out.reshape(b, t, W_E.shape[1])
